# baseline (speedup 1.0000x reference)
import functools

import jax
import jax.numpy as jnp
from jax import lax
from jax.experimental import pallas as pl
from jax.experimental.pallas import tpu as pltpu
from jax.experimental.pallas import tpu_sc as plsc

G = 8
_NC, _NS = 2, 16
_NW = _NC * _NS


def _sc_mesh():
    return plsc.VectorSubcoreMesh(
        core_axis_name="c", subcore_axis_name="s",
        num_cores=_NC, num_subcores=_NS)


def _routing_body(x_ref, rw_ref, sidx1_ref, gate1_ref, sidx2_ref, gate2_ref,
                  cidx1_ref, cidx2_ref, xb_ref):
    S, _ = x_ref.shape
    E = rw_ref.shape[1]
    C = S // E
    g = pl.program_id(0)

    xb_ref[...] = x_ref[...].astype(jnp.bfloat16)[None]
    logits = jnp.dot(x_ref[...], rw_ref[...])
    m = jnp.max(logits, axis=-1, keepdims=True)
    ex = jnp.exp(logits - m)
    raw = ex / jnp.sum(ex, axis=-1, keepdims=True)

    e_iota = lax.broadcasted_iota(jnp.int32, (S, E), 1)

    gate1 = jnp.max(raw, axis=-1)
    idx1 = jnp.min(jnp.where(raw == gate1[:, None], e_iota, E), axis=-1)
    mask1 = (e_iota == idx1[:, None]).astype(jnp.float32)

    raw2 = raw * (1.0 - mask1)
    gate2 = jnp.max(raw2, axis=-1)
    idx2 = jnp.min(jnp.where(raw2 == gate2[:, None], e_iota, E), axis=-1)
    mask2 = (e_iota == idx2[:, None]).astype(jnp.float32)

    r_iota = lax.broadcasted_iota(jnp.int32, (S, S), 0)
    c_iota = lax.broadcasted_iota(jnp.int32, (S, S), 1)
    tril = (r_iota > c_iota).astype(jnp.float32)
    pos1_all = jnp.dot(tril, mask1)
    keep1 = (pos1_all < C) & (mask1 > 0.0)
    mask1c = jnp.where(keep1, 1.0, 0.0)
    pos1 = jnp.sum(pos1_all * mask1c, axis=-1)
    kept1 = jnp.sum(mask1c, axis=-1)
    count1 = jnp.sum(mask1c, axis=0)

    pos2_all = jnp.dot(tril, mask2) + count1[None, :]
    keep2 = (pos2_all < C) & (mask2 > 0.0)
    mask2c = jnp.where(keep2, 1.0, 0.0)
    pos2 = jnp.sum(pos2_all * mask2c, axis=-1)
    kept2 = jnp.sum(mask2c, axis=-1)

    gate1 = gate1 * kept1
    gate2 = gate2 * kept2

    trash = E * G * C
    slot1 = idx1 * (G * C) + g * C + pos1.astype(jnp.int32)
    slot2 = idx2 * (G * C) + g * C + pos2.astype(jnp.int32)
    k1 = kept1 > 0.0
    k2 = kept2 > 0.0
    sidx1_ref[...] = jnp.where(k1, slot1, trash)[None, None, :]
    sidx2_ref[...] = jnp.where(k2, slot2, trash)[None, None, :]
    cidx1_ref[...] = jnp.where(k1, slot1, 0)[None, None, :]
    cidx2_ref[...] = jnp.where(k2, slot2, 0)[None, None, :]
    gate1_ref[...] = jnp.broadcast_to(gate1[:, None], (S, 16))[None]
    gate2_ref[...] = jnp.broadcast_to(gate2[:, None], (S, 16))[None]


def _routing_call(x, router_w, interpret=False):
    G_, S, M = x.shape
    E = router_w.shape[1]
    i32 = jax.ShapeDtypeStruct((G_, 1, S), jnp.int32)
    f32e = jax.ShapeDtypeStruct((G_, S, 16), jnp.float32)
    xb16 = jax.ShapeDtypeStruct((G_, S, M), jnp.bfloat16)

    def body(x_ref, rw_ref, s1, g1, s2, g2, c1, c2, xb):
        _routing_body(x_ref[0], rw_ref, s1, g1, s2, g2, c1, c2, xb)

    idx_spec = pl.BlockSpec((1, 1, S), lambda g: (g, 0, 0))
    gate_spec = pl.BlockSpec((1, S, 16), lambda g: (g, 0, 0))
    return pl.pallas_call(
        body,
        grid=(G_,),
        in_specs=[
            pl.BlockSpec((1, S, M), lambda g: (g, 0, 0)),
            pl.BlockSpec((M, E), lambda g: (0, 0)),
        ],
        out_specs=[idx_spec, gate_spec, idx_spec, gate_spec, idx_spec,
                   idx_spec, pl.BlockSpec((1, S, M), lambda g: (g, 0, 0))],
        out_shape=[i32, f32e, i32, f32e, i32, i32, xb16],
        interpret=interpret,
    )(x, router_w)


def _ffn(ei_flat, w0, w1, wo, *, hc=1024, interpret=False):
    E, M, H = w0.shape
    R = 256
    grid = (E, H // hc)

    def body(ei_ref, w0_ref, w1_ref, wo_ref, out_ref):
        h = pl.program_id(1)
        a = ei_ref[...].astype(jnp.float32)
        h0 = jnp.dot(a, w0_ref[0])
        h1 = jnp.dot(a, w1_ref[0])
        part = jnp.dot(jax.nn.gelu(h0) * h1, wo_ref[0])

        @pl.when(h == 0)
        def _():
            out_ref[...] = part

        @pl.when(h > 0)
        def _():
            out_ref[...] += part

    return pl.pallas_call(
        body,
        grid=grid,
        in_specs=[
            pl.BlockSpec((R, M), lambda e, h: (e, 0)),
            pl.BlockSpec((1, M, hc), lambda e, h: (e, 0, h)),
            pl.BlockSpec((1, M, hc), lambda e, h: (e, 0, h)),
            pl.BlockSpec((1, hc, M), lambda e, h: (e, h, 0)),
        ],
        out_specs=pl.BlockSpec((R, M), lambda e, h: (e, 0)),
        out_shape=jax.ShapeDtypeStruct((E * R, M), jnp.float32),
        compiler_params=pltpu.CompilerParams(
            dimension_semantics=("parallel", "arbitrary"),
        ),
        interpret=interpret,
    )(ei_flat, w0, w1, wo)


def _dispatch_sc(x_flat, sidx1, sidx2, n_slots):
    T, M = x_flat.shape
    TPW = T // _NW
    CK = 64
    NCH = TPW // CK

    NB = 2

    @functools.partial(
        pl.kernel,
        mesh=_sc_mesh(),
        out_type=jax.ShapeDtypeStruct((n_slots, M), jnp.int32),
        scratch_types=[
            pltpu.VMEM((NB, CK, M), jnp.int32),
            pltpu.VMEM((NB, CK), jnp.int32),
            pltpu.VMEM((NB, CK), jnp.int32),
            pltpu.SemaphoreType.DMA,
            pltpu.SemaphoreType.DMA,
        ],
    )
    def k(x_hbm, i1_hbm, i2_hbm, ei_hbm, xbuf, i1v, i2v, sem0, sem1):
        wid = lax.axis_index("s") * _NC + lax.axis_index("c")
        base = wid * TPW
        sems = [sem0, sem1]
        pend = [None] * NB
        for c in range(NCH):
            b = c % NB
            if pend[b] is not None:
                pend[b][0].wait()
                pend[b][1].wait()
            off = base + c * CK
            pltpu.sync_copy(x_hbm.at[pl.ds(off, CK)], xbuf.at[b])
            pltpu.sync_copy(i1_hbm.at[pl.ds(off, CK)], i1v.at[b])
            pltpu.sync_copy(i2_hbm.at[pl.ds(off, CK)], i2v.at[b])
            h1 = pltpu.async_copy(xbuf.at[b], ei_hbm.at[i1v.at[b]], sems[b])
            h2 = pltpu.async_copy(xbuf.at[b], ei_hbm.at[i2v.at[b]], sems[b])
            pend[b] = (h1, h2)
        for b in range(NB):
            if pend[b] is not None:
                pend[b][0].wait()
                pend[b][1].wait()

    return k(x_flat, sidx1, sidx2)


def _combine_sc(eo_flat, cidx1, gate1, cidx2, gate2):
    T, M = eo_flat.shape
    TPW = T // _NW
    CK = 32
    NCH = TPW // CK
    NB = 2

    @functools.partial(
        pl.kernel,
        mesh=_sc_mesh(),
        out_type=jax.ShapeDtypeStruct((T, M), jnp.float32),
        scratch_types=[
            pltpu.VMEM((NB, CK, M), jnp.float32),
            pltpu.VMEM((NB, CK, M), jnp.float32),
            pltpu.VMEM((NB, CK), jnp.int32),
            pltpu.VMEM((NB, CK), jnp.int32),
            pltpu.VMEM((NB, CK, 16), jnp.float32),
            pltpu.VMEM((NB, CK, 16), jnp.float32),
            pltpu.SemaphoreType.DMA,
            pltpu.SemaphoreType.DMA,
        ],
    )
    def k(eo_hbm, i1_hbm, g1_hbm, i2_hbm, g2_hbm, out_hbm,
          buf1, buf2, i1v, i2v, g1v, g2v, gsem, osem):
        wid = lax.axis_index("s") * _NC + lax.axis_index("c")
        base = wid * TPW
        zero = jnp.zeros((16,), jnp.float32)

        def load_and_fire(c):
            b = c % NB
            off = base + c * CK
            pltpu.sync_copy(i1_hbm.at[pl.ds(off, CK)], i1v.at[b])
            pltpu.sync_copy(i2_hbm.at[pl.ds(off, CK)], i2v.at[b])
            pltpu.sync_copy(g1_hbm.at[pl.ds(off, CK)], g1v.at[b])
            pltpu.sync_copy(g2_hbm.at[pl.ds(off, CK)], g2v.at[b])
            h1 = pltpu.async_copy(eo_hbm.at[i1v.at[b]], buf1.at[b], gsem)
            h2 = pltpu.async_copy(eo_hbm.at[i2v.at[b]], buf2.at[b], gsem)
            return (h1, h2)

        gpend = [None] * NB
        opend = [None] * NB
        gpend[0] = load_and_fire(0)
        for c in range(NCH):
            b = c % NB
            nb_ = (c + 1) % NB
            if c + 1 < NCH:
                if opend[nb_] is not None:
                    opend[nb_].wait()
                    opend[nb_] = None
                gpend[nb_] = load_and_fire(c + 1)
            gpend[b][0].wait()
            gpend[b][1].wait()

            def body(j, carry):
                gav = g1v[b, j, :]
                gbv = g2v[b, j, :]
                ma = gav > 0.0
                mb = gbv > 0.0
                for kk in range(M // 16):
                    sl = pl.ds(kk * 16, 16)
                    r1 = buf1[b, j, sl]
                    r2 = buf2[b, j, sl]
                    buf1[b, j, sl] = (jnp.where(ma, r1 * gav, zero)
                                      + jnp.where(mb, r2 * gbv, zero))
                return carry

            lax.fori_loop(0, CK, body, 0)
            off = base + c * CK
            opend[b] = pltpu.async_copy(
                buf1.at[b], out_hbm.at[pl.ds(off, CK)], osem)
        for b in range(NB):
            if opend[b] is not None:
                opend[b].wait()

    return k(eo_flat, cidx1, gate1, cidx2, gate2)


def kernel(inputs, router_w, w0, w1, wo):
    B, L, M = inputs.shape
    E = router_w.shape[1]
    S = B * L // G
    C = S // E
    x = inputs.reshape(G, S, M)

    sidx1, gate1, sidx2, gate2, cidx1, cidx2, xb = _routing_call(x, router_w)

    xb_i32 = jax.lax.bitcast_convert_type(
        xb.reshape(G * S, M // 2, 2), jnp.int32)
    n_slots = E * G * C + 8
    ei = _dispatch_sc(xb_i32, sidx1.reshape(-1), sidx2.reshape(-1), n_slots)
    ei_bf16 = jax.lax.bitcast_convert_type(ei, jnp.bfloat16).reshape(
        n_slots, M)
    eo_flat = _ffn(ei_bf16, w0, w1, wo)
    out = _combine_sc(eo_flat, cidx1.reshape(-1), gate1.reshape(G * S, 16),
                      cidx2.reshape(-1), gate2.reshape(G * S, 16))
    return out.reshape(B, L, M)


# speedup vs baseline: 3.1759x; 3.1759x over previous
import jax
import jax.numpy as jnp
from jax import lax
from jax.experimental import pallas as pl
from jax.experimental.pallas import tpu as pltpu

G = 8


def _routing_dispatch_body(x_ref, rw_ref, ei_ref, cs1_ref, g1_ref, cs2_ref,
                           g2_ref):
    _, S, MM = x_ref.shape
    E = rw_ref.shape[1]
    C = S // E
    P = E * C

    x = x_ref[0]
    logits = jnp.dot(x, rw_ref[...])
    m = jnp.max(logits, axis=-1, keepdims=True)
    ex = jnp.exp(logits - m)
    raw = ex / jnp.sum(ex, axis=-1, keepdims=True)

    e_iota = lax.broadcasted_iota(jnp.int32, (S, E), 1)

    gate1 = jnp.max(raw, axis=-1)
    idx1 = jnp.min(jnp.where(raw == gate1[:, None], e_iota, E), axis=-1)
    mask1 = (e_iota == idx1[:, None]).astype(jnp.float32)

    raw2 = raw * (1.0 - mask1)
    gate2 = jnp.max(raw2, axis=-1)
    idx2 = jnp.min(jnp.where(raw2 == gate2[:, None], e_iota, E), axis=-1)
    mask2 = (e_iota == idx2[:, None]).astype(jnp.float32)

    r_iota = lax.broadcasted_iota(jnp.int32, (S, S), 0)
    c_iota = lax.broadcasted_iota(jnp.int32, (S, S), 1)
    tril = (r_iota > c_iota).astype(jnp.float32)
    pos1_all = jnp.dot(tril, mask1)
    keep1 = (pos1_all < C) & (mask1 > 0.0)
    mask1c = jnp.where(keep1, 1.0, 0.0)
    pos1 = jnp.sum(pos1_all * mask1c, axis=-1)
    kept1 = jnp.sum(mask1c, axis=-1)
    count1 = jnp.sum(mask1c, axis=0)

    pos2_all = jnp.dot(tril, mask2) + count1[None, :]
    keep2 = (pos2_all < C) & (mask2 > 0.0)
    mask2c = jnp.where(keep2, 1.0, 0.0)
    pos2 = jnp.sum(pos2_all * mask2c, axis=-1)
    kept2 = jnp.sum(mask2c, axis=-1)

    k1 = kept1 > 0.0
    k2 = kept2 > 0.0
    slot1 = jnp.where(k1, idx1 * C + pos1.astype(jnp.int32), -1)
    slot2 = jnp.where(k2, idx2 * C + pos2.astype(jnp.int32), -1)

    p_iota = lax.broadcasted_iota(jnp.int32, (S, P), 1)
    disp = ((p_iota == slot1[:, None]) | (p_iota == slot2[:, None]))
    disp = disp.astype(jnp.float32)
    ei_ref[0] = lax.dot_general(
        disp, x, dimension_numbers=(((0,), (0,)), ((), ())))

    cs1_ref[...] = slot1[None, None, :]
    cs2_ref[...] = slot2[None, None, :]
    g1_ref[...] = (gate1 * kept1)[None, None, :]
    g2_ref[...] = (gate2 * kept2)[None, None, :]


def _routing_dispatch(x, router_w, interpret=False):
    G_, S, M = x.shape
    E = router_w.shape[1]
    P = S
    i32 = jax.ShapeDtypeStruct((G_, 1, S), jnp.int32)
    f32 = jax.ShapeDtypeStruct((G_, 1, S), jnp.float32)
    ei = jax.ShapeDtypeStruct((G_, P, M), jnp.float32)
    sl_spec = pl.BlockSpec((1, 1, S), lambda g: (g, 0, 0))
    return pl.pallas_call(
        _routing_dispatch_body,
        grid=(G_,),
        in_specs=[
            pl.BlockSpec((1, S, M), lambda g: (g, 0, 0)),
            pl.BlockSpec((M, E), lambda g: (0, 0)),
        ],
        out_specs=[pl.BlockSpec((1, P, M), lambda g: (g, 0, 0)),
                   sl_spec, sl_spec, sl_spec, sl_spec],
        out_shape=[ei, i32, f32, i32, f32],
        interpret=interpret,
    )(x, router_w)


def _ffn(ei, w0, w1, wo, *, hc=1024, interpret=False):
    E, M, H = w0.shape
    G_, P, _ = ei.shape
    C = P // E
    grid = (E, H // hc)

    def body(ei_ref, w0_ref, w1_ref, wo_ref, out_ref):
        h = pl.program_id(1)
        a = ei_ref[...].reshape(G_ * C, M)
        h0 = jnp.dot(a, w0_ref[0])
        h1 = jnp.dot(a, w1_ref[0])
        part = jnp.dot(jax.nn.gelu(h0) * h1, wo_ref[0])

        @pl.when(h == 0)
        def _():
            out_ref[...] = part.reshape(1, G_, C, M)

        @pl.when(h > 0)
        def _():
            out_ref[...] += part.reshape(1, G_, C, M)

    return pl.pallas_call(
        body,
        grid=grid,
        in_specs=[
            pl.BlockSpec((G_, C, M), lambda e, h: (0, e, 0)),
            pl.BlockSpec((1, M, hc), lambda e, h: (e, 0, h)),
            pl.BlockSpec((1, M, hc), lambda e, h: (e, 0, h)),
            pl.BlockSpec((1, hc, M), lambda e, h: (e, h, 0)),
        ],
        out_specs=pl.BlockSpec((1, G_, C, M), lambda e, h: (e, 0, 0, 0)),
        out_shape=jax.ShapeDtypeStruct((E, G_, C, M), jnp.float32),
        compiler_params=pltpu.CompilerParams(
            dimension_semantics=("parallel", "arbitrary"),
        ),
        interpret=interpret,
    )(ei, w0, w1, wo)


def _combine(eo, cs1, g1, cs2, g2, interpret=False):
    E, G_, C, M = eo.shape
    S = cs1.shape[2]
    P = E * C

    def body(eo_ref, cs1_ref, g1_ref, cs2_ref, g2_ref, out_ref):
        eo_mat = eo_ref[...].reshape(P, M)
        s1 = cs1_ref[0, 0, :]
        s2 = cs2_ref[0, 0, :]
        ga = g1_ref[0, 0, :]
        gb = g2_ref[0, 0, :]
        p_iota = lax.broadcasted_iota(jnp.int32, (S, P), 1)
        cmb = (jnp.where(p_iota == s1[:, None], ga[:, None], 0.0)
               + jnp.where(p_iota == s2[:, None], gb[:, None], 0.0))
        out_ref[0] = jnp.dot(cmb, eo_mat)

    sl_spec = pl.BlockSpec((1, 1, S), lambda g: (g, 0, 0))
    return pl.pallas_call(
        body,
        grid=(G_,),
        in_specs=[
            pl.BlockSpec((E, 1, C, M), lambda g: (0, g, 0, 0)),
            sl_spec, sl_spec, sl_spec, sl_spec,
        ],
        out_specs=pl.BlockSpec((1, S, M), lambda g: (g, 0, 0)),
        out_shape=jax.ShapeDtypeStruct((G_, S, M), jnp.float32),
        interpret=interpret,
    )(eo, cs1, g1, cs2, g2)


def kernel(inputs, router_w, w0, w1, wo):
    B, L, M = inputs.shape
    S = B * L // G
    x = inputs.reshape(G, S, M)

    ei, cs1, g1, cs2, g2 = _routing_dispatch(x, router_w)
    eo = _ffn(ei, w0, w1, wo)
    out = _combine(eo, cs1, g1, cs2, g2)
    return out.reshape(B, L, M)
